# baseline (speedup 1.0000x reference)
import functools

import jax
import jax.numpy as jnp
from jax import lax
from jax.experimental import pallas as pl
from jax.experimental.pallas import tpu as pltpu
from jax.experimental.pallas import tpu_sc as plsc

N_NODES = 100000
N_EDGES = 6400000
NC = 2
NS = 16
LANE = 128
N_PAD = 100352
SUB = N_PAD // NS
HALF = SUB // 2
GROUP = 128
GPC = 80
CHUNK = GROUP * GPC
N_CHUNKS = N_EDGES // CHUNK
BASE_CH = N_CHUNKS // (NC * NS)
EXTRA = N_CHUNKS - BASE_CH * NC * NS

_mesh = plsc.VectorSubcoreMesh(
    core_axis_name="c", subcore_axis_name="s", num_cores=NC, num_subcores=NS)


def _drain(n, ref_hbm, dst, sem):
    for _ in range(n):
        pltpu.make_async_copy(ref_hbm, dst, sem).wait()


@functools.partial(
    pl.kernel,
    out_type=jax.ShapeDtypeStruct((NC, N_PAD), jnp.float32),
    mesh=_mesh,
    scratch_types=[
        pltpu.VMEM_SHARED((N_PAD,), jnp.float32),
        pltpu.VMEM((GPC, GROUP), jnp.int32),
        pltpu.VMEM((GPC, GROUP), jnp.int32),
        pltpu.VMEM((GROUP,), jnp.float32),
        pltpu.SemaphoreType.DMA,
        pltpu.SemaphoreType.DMA,
    ],
)
def _deg_kernel(col_hbm, ones_hbm, zeros_hbm, degp_hbm,
                deg_sh, idx_a, idx_b, ones_v, sem_i, sem_s):
    c = lax.axis_index("c")
    s = lax.axis_index("s")
    w = c * NS + s
    pltpu.sync_copy(ones_hbm, ones_v)
    pltpu.sync_copy(zeros_hbm, deg_sh.at[pl.ds(s * SUB, SUB)])
    plsc.subcore_barrier()

    nch = jnp.where(w < EXTRA, BASE_CH + 1, BASE_CH)
    cbase = w * BASE_CH + jnp.minimum(w, EXTRA)

    pltpu.sync_copy(col_hbm.at[pl.ds(cbase * GPC, GPC), :], idx_a)

    def step(k, idx_p, idx_q):
        @pl.when(k > 0)
        def _():
            _drain(1, col_hbm.at[pl.ds(0, GPC), :], idx_p, sem_i)

        for j in range(GPC):
            pltpu.async_copy(ones_v, deg_sh.at[idx_p.at[j]], sem_s, add=True)

        @pl.when(k > 0)
        def _():
            _drain(GPC, ones_hbm, idx_q.at[0], sem_s)

        @pl.when(k < nch - 1)
        def _():
            pltpu.async_copy(col_hbm.at[pl.ds((cbase + k + 1) * GPC, GPC), :],
                             idx_q, sem_i)

    def chunk_body(k, carry):
        @pl.when(k % 2 == 0)
        def _():
            step(k, idx_a, idx_b)

        @pl.when(k % 2 == 1)
        def _():
            step(k, idx_b, idx_a)

        return carry

    lax.fori_loop(0, nch, chunk_body, 0)
    _drain(GPC, ones_hbm, idx_a.at[0], sem_s)
    plsc.subcore_barrier()
    pltpu.sync_copy(deg_sh.at[pl.ds(s * SUB, SUB)],
                    degp_hbm.at[c, pl.ds(s * SUB, SUB)])


@functools.partial(
    pl.kernel,
    out_type=(jax.ShapeDtypeStruct((NC, N_PAD), jnp.float32),
              jax.ShapeDtypeStruct((NC, N_PAD), jnp.float32)),
    mesh=_mesh,
    scratch_types=[
        pltpu.VMEM_SHARED((N_PAD,), jnp.float32),
        pltpu.VMEM_SHARED((N_PAD,), jnp.float32),
        pltpu.VMEM_SHARED((N_PAD,), jnp.float32),
        pltpu.VMEM_SHARED((N_PAD,), jnp.float32),
        pltpu.VMEM((GPC, GROUP), jnp.int32),
        pltpu.VMEM((GPC, GROUP), jnp.int32),
        pltpu.VMEM((GPC, GROUP), jnp.int32),
        pltpu.VMEM((GPC, GROUP), jnp.int32),
        pltpu.VMEM((GPC, GROUP), jnp.float32),
        pltpu.VMEM((GPC, GROUP), jnp.float32),
        pltpu.VMEM((GPC, GROUP), jnp.float32),
        pltpu.VMEM((GPC, GROUP), jnp.float32),
        pltpu.VMEM((HALF,), jnp.float32),
        pltpu.VMEM((HALF,), jnp.float32),
        pltpu.VMEM((HALF,), jnp.float32),
        pltpu.VMEM((HALF,), jnp.float32),
        pltpu.VMEM((HALF,), jnp.float32),
        pltpu.VMEM((HALF,), jnp.float32),
        pltpu.VMEM((HALF,), jnp.float32),
        pltpu.VMEM((6 * 128,), jnp.float32),
        pltpu.SemaphoreType.DMA,
        pltpu.SemaphoreType.DMA,
        pltpu.SemaphoreType.DMA,
    ],
    compiler_params=pltpu.CompilerParams(needs_layout_passes=False),
)
def _msg_kernel(row_hbm, col_hbm, dp0_hbm, dp1_hbm, x0_hbm, x1_hbm, x2_hbm,
                wt_hbm, zeros_hbm,
                acc0_hbm, acc1_hbm,
                g0_sh, g1_sh, acc0_sh, acc1_sh,
                row_a, row_b, col_a, col_b, v0a, v0b, v1a, v1b,
                p0_v, p1_v, x0_v, x1_v, x2_v, g0_v, g1_v, wt_v,
                sem_i, sem_g, sem_s):
    c = lax.axis_index("c")
    s = lax.axis_index("s")
    w = c * NS + s
    sl = pl.ds(s * SUB, SUB)
    pltpu.sync_copy(zeros_hbm, acc0_sh.at[sl])
    pltpu.sync_copy(zeros_hbm, acc1_sh.at[sl])
    pltpu.sync_copy(wt_hbm, wt_v)
    for hh in range(2):
        off = s * SUB + hh * HALF
        pltpu.sync_copy(dp0_hbm.at[pl.ds(off, HALF)], p0_v)
        pltpu.sync_copy(dp1_hbm.at[pl.ds(off, HALF)], p1_v)
        pltpu.sync_copy(x0_hbm.at[pl.ds(off, HALF)], x0_v)
        pltpu.sync_copy(x1_hbm.at[pl.ds(off, HALF)], x1_v)
        pltpu.sync_copy(x2_hbm.at[pl.ds(off, HALF)], x2_v)

        def g_body(i, carry):
            ix = pl.ds(i * 16, 16)
            d = p0_v[ix] + p1_v[ix] + 1.0
            bits = plsc.bitcast(d, jnp.int32)
            y = plsc.bitcast(jnp.int32(0x5F3759DF) - (bits >> 1), jnp.float32)
            for _ in range(3):
                y = y * (1.5 - 0.5 * d * y * y)
            x0 = x0_v[ix]
            x1 = x1_v[ix]
            x2 = x2_v[ix]
            g0_v[ix] = (x0 * wt_v[pl.ds(0, 16)] + x1 * wt_v[pl.ds(128, 16)]
                        + x2 * wt_v[pl.ds(256, 16)]) * y
            g1_v[ix] = (x0 * wt_v[pl.ds(384, 16)] + x1 * wt_v[pl.ds(512, 16)]
                        + x2 * wt_v[pl.ds(640, 16)]) * y
            return carry

        lax.fori_loop(0, HALF // 16, g_body, 0)
        pltpu.sync_copy(g0_v, g0_sh.at[pl.ds(off, HALF)])
        pltpu.sync_copy(g1_v, g1_sh.at[pl.ds(off, HALF)])
    plsc.subcore_barrier()

    nch = jnp.where(w < EXTRA, BASE_CH + 1, BASE_CH)
    cbase = w * BASE_CH + jnp.minimum(w, EXTRA)

    pltpu.sync_copy(row_hbm.at[pl.ds(cbase * GPC, GPC), :], row_a)
    pltpu.sync_copy(col_hbm.at[pl.ds(cbase * GPC, GPC), :], col_a)

    def step(k, row_p, col_p, v0p, v1p, row_q, col_q, v0q, v1q):
        @pl.when(k > 0)
        def _():
            _drain(2, row_hbm.at[pl.ds(0, GPC), :], row_p, sem_i)

        for j in range(GPC):
            pltpu.async_copy(g0_sh.at[row_p.at[j]], v0p.at[j], sem_g)
        for j in range(GPC):
            pltpu.async_copy(g1_sh.at[row_p.at[j]], v1p.at[j], sem_g)

        @pl.when(k > 0)
        def _():
            _drain(2 * GPC, zeros_hbm, v0q.at[0], sem_s)

        @pl.when(k < nch - 1)
        def _():
            nxt = pl.ds((cbase + k + 1) * GPC, GPC)
            pltpu.async_copy(row_hbm.at[nxt, :], row_q, sem_i)
            pltpu.async_copy(col_hbm.at[nxt, :], col_q, sem_i)

        _drain(2 * GPC, zeros_hbm, v0p.at[0], sem_g)

        for j in range(GPC):
            pltpu.async_copy(v0p.at[j], acc0_sh.at[col_p.at[j]], sem_s,
                             add=True)
        for j in range(GPC):
            pltpu.async_copy(v1p.at[j], acc1_sh.at[col_p.at[j]], sem_s,
                             add=True)

    def chunk_body(k, carry):
        @pl.when(k % 2 == 0)
        def _():
            step(k, row_a, col_a, v0a, v1a, row_b, col_b, v0b, v1b)

        @pl.when(k % 2 == 1)
        def _():
            step(k, row_b, col_b, v0b, v1b, row_a, col_a, v0a, v1a)

        return carry

    lax.fori_loop(0, nch, chunk_body, 0)
    _drain(2 * GPC, zeros_hbm, v0a.at[0], sem_s)
    plsc.subcore_barrier()
    pltpu.sync_copy(acc0_sh.at[sl], acc0_hbm.at[c, sl])
    pltpu.sync_copy(acc1_sh.at[sl], acc1_hbm.at[c, sl])


_R = N_PAD // LANE


def _combine_body(acc0_ref, acc1_ref, degp_ref, x_ref, w_ref, b_ref, o_ref):
    deg = degp_ref[0] + degp_ref[1] + 1.0
    dinv = lax.rsqrt(deg)
    x0, x1, x2 = x_ref[0], x_ref[1], x_ref[2]
    g0 = (x0 * w_ref[0, 0] + x1 * w_ref[1, 0] + x2 * w_ref[2, 0]) * dinv
    g1 = (x0 * w_ref[0, 1] + x1 * w_ref[1, 1] + x2 * w_ref[2, 1]) * dinv
    o_ref[0] = dinv * (acc0_ref[0] + acc0_ref[1] + g0) + b_ref[0]
    o_ref[1] = dinv * (acc1_ref[0] + acc1_ref[1] + g1) + b_ref[1]


_combine = pl.pallas_call(
    _combine_body,
    out_shape=jax.ShapeDtypeStruct((2, _R, LANE), jnp.float32),
    in_specs=[pl.BlockSpec(memory_space=pltpu.VMEM),
              pl.BlockSpec(memory_space=pltpu.VMEM),
              pl.BlockSpec(memory_space=pltpu.VMEM),
              pl.BlockSpec(memory_space=pltpu.VMEM),
              pl.BlockSpec(memory_space=pltpu.SMEM),
              pl.BlockSpec(memory_space=pltpu.SMEM)],
    out_specs=pl.BlockSpec(memory_space=pltpu.VMEM),
)


def kernel(x, edge_index, W, b):
    ei = edge_index.astype(jnp.int32)
    row = ei[0].reshape(N_EDGES // GROUP, GROUP)
    col = ei[1].reshape(N_EDGES // GROUP, GROUP)
    ones_g = jnp.ones((GROUP,), jnp.float32)
    zeros_s = jnp.zeros((SUB,), jnp.float32)

    degp = _deg_kernel(col, ones_g, zeros_s)

    Wf = W.astype(jnp.float32)
    x_pad = jnp.pad(x.astype(jnp.float32).T, ((0, 0), (0, N_PAD - N_NODES)))
    wt = jnp.repeat(Wf.T.reshape(6), 128)

    acc0, acc1 = _msg_kernel(row, col, degp[0], degp[1],
                             x_pad[0], x_pad[1], x_pad[2], wt, zeros_s)

    outT = _combine(acc0.reshape(NC, _R, LANE), acc1.reshape(NC, _R, LANE),
                    degp.reshape(NC, _R, LANE), x_pad.reshape(3, _R, LANE),
                    Wf, b.astype(jnp.float32))
    return outT.reshape(2, N_PAD)[:, :N_NODES].T


# speedup vs baseline: 1.0485x; 1.0485x over previous
import functools

import jax
import jax.numpy as jnp
from jax import lax
from jax.experimental import pallas as pl
from jax.experimental.pallas import tpu as pltpu
from jax.experimental.pallas import tpu_sc as plsc

N_NODES = 100000
N_EDGES = 6400000
NC = 2
NS = 16
LANE = 128
N_PAD = 100352
SUB = N_PAD // NS
GROUP = 128
GPC = 80
CHUNK = GROUP * GPC
N_CHUNKS = N_EDGES // CHUNK
BASE_CH = N_CHUNKS // (NC * NS)
EXTRA = N_CHUNKS - BASE_CH * NC * NS

_mesh = plsc.VectorSubcoreMesh(
    core_axis_name="c", subcore_axis_name="s", num_cores=NC, num_subcores=NS)


def _drain(n, ref_hbm, dst, sem):
    for _ in range(n):
        pltpu.make_async_copy(ref_hbm, dst, sem).wait()


@functools.partial(
    pl.kernel,
    out_type=jax.ShapeDtypeStruct((NC, N_PAD), jnp.float32),
    mesh=_mesh,
    scratch_types=[
        pltpu.VMEM_SHARED((N_PAD,), jnp.float32),
        pltpu.VMEM((GPC, GROUP), jnp.int32),
        pltpu.VMEM((GPC, GROUP), jnp.int32),
        pltpu.VMEM((GROUP,), jnp.float32),
        pltpu.SemaphoreType.DMA,
        pltpu.SemaphoreType.DMA,
    ],
)
def _deg_kernel(col_hbm, ones_hbm, zeros_hbm, degp_hbm,
                deg_sh, idx_a, idx_b, ones_v, sem_i, sem_s):
    c = lax.axis_index("c")
    s = lax.axis_index("s")
    w = c * NS + s
    pltpu.sync_copy(ones_hbm, ones_v)
    pltpu.sync_copy(zeros_hbm, deg_sh.at[pl.ds(s * SUB, SUB)])
    plsc.subcore_barrier()

    nch = jnp.where(w < EXTRA, BASE_CH + 1, BASE_CH)
    cbase = w * BASE_CH + jnp.minimum(w, EXTRA)

    pltpu.sync_copy(col_hbm.at[pl.ds(cbase * GPC, GPC), :], idx_a)

    def step(k, idx_p, idx_q):
        @pl.when(k > 0)
        def _():
            _drain(1, col_hbm.at[pl.ds(0, GPC), :], idx_p, sem_i)

        for j in range(GPC):
            pltpu.async_copy(ones_v, deg_sh.at[idx_p.at[j]], sem_s, add=True)

        @pl.when(k > 0)
        def _():
            _drain(1, col_hbm.at[pl.ds(0, GPC), :], idx_q, sem_s)

        @pl.when(k < nch - 1)
        def _():
            pltpu.async_copy(col_hbm.at[pl.ds((cbase + k + 1) * GPC, GPC), :],
                             idx_q, sem_i)

    def chunk_body(k, carry):
        @pl.when(k % 2 == 0)
        def _():
            step(k, idx_a, idx_b)

        @pl.when(k % 2 == 1)
        def _():
            step(k, idx_b, idx_a)

        return carry

    lax.fori_loop(0, nch, chunk_body, 0)
    _drain(1, col_hbm.at[pl.ds(0, GPC), :], idx_a, sem_s)
    plsc.subcore_barrier()
    pltpu.sync_copy(deg_sh.at[pl.ds(s * SUB, SUB)],
                    degp_hbm.at[c, pl.ds(s * SUB, SUB)])


@functools.partial(
    pl.kernel,
    out_type=(jax.ShapeDtypeStruct((NC, N_PAD), jnp.float32),
              jax.ShapeDtypeStruct((NC, N_PAD), jnp.float32)),
    mesh=_mesh,
    scratch_types=[
        pltpu.VMEM_SHARED((N_PAD,), jnp.float32),
        pltpu.VMEM_SHARED((N_PAD,), jnp.float32),
        pltpu.VMEM_SHARED((N_PAD,), jnp.float32),
        pltpu.VMEM_SHARED((N_PAD,), jnp.float32),
        pltpu.VMEM((GPC, GROUP), jnp.int32),
        pltpu.VMEM((GPC, GROUP), jnp.int32),
        pltpu.VMEM((GPC, GROUP), jnp.int32),
        pltpu.VMEM((GPC, GROUP), jnp.int32),
        pltpu.VMEM((GPC, GROUP), jnp.float32),
        pltpu.VMEM((GPC, GROUP), jnp.float32),
        pltpu.VMEM((GPC, GROUP), jnp.float32),
        pltpu.VMEM((GPC, GROUP), jnp.float32),
        pltpu.SemaphoreType.DMA,
        pltpu.SemaphoreType.DMA,
        pltpu.SemaphoreType.DMA,
    ],
)
def _msg_kernel(row_hbm, col_hbm, g0_hbm, g1_hbm, zeros_hbm,
                acc0_hbm, acc1_hbm,
                g0_sh, g1_sh, acc0_sh, acc1_sh,
                row_a, row_b, col_a, col_b, v0a, v0b, v1a, v1b,
                sem_i, sem_g, sem_s):
    c = lax.axis_index("c")
    s = lax.axis_index("s")
    w = c * NS + s
    sl = pl.ds(s * SUB, SUB)
    pltpu.sync_copy(zeros_hbm, acc0_sh.at[sl])
    pltpu.sync_copy(zeros_hbm, acc1_sh.at[sl])
    pltpu.sync_copy(g0_hbm.at[sl], g0_sh.at[sl])
    pltpu.sync_copy(g1_hbm.at[sl], g1_sh.at[sl])
    plsc.subcore_barrier()

    nch = jnp.where(w < EXTRA, BASE_CH + 1, BASE_CH)
    cbase = w * BASE_CH + jnp.minimum(w, EXTRA)

    pltpu.sync_copy(row_hbm.at[pl.ds(cbase * GPC, GPC), :], row_a)
    pltpu.sync_copy(col_hbm.at[pl.ds(cbase * GPC, GPC), :], col_a)

    def step(k, row_p, col_p, v0p, v1p, row_q, col_q, v0q, v1q):
        @pl.when(k > 0)
        def _():
            _drain(2, row_hbm.at[pl.ds(0, GPC), :], row_p, sem_i)

        for j in range(GPC):
            pltpu.async_copy(g0_sh.at[row_p.at[j]], v0p.at[j], sem_g)
        for j in range(GPC):
            pltpu.async_copy(g1_sh.at[row_p.at[j]], v1p.at[j], sem_g)

        @pl.when(k > 0)
        def _():
            _drain(1, row_hbm.at[pl.ds(0, GPC), :].bitcast(jnp.float32), v0q,
                   sem_s)
            _drain(1, row_hbm.at[pl.ds(0, GPC), :].bitcast(jnp.float32), v1q,
                   sem_s)

        @pl.when(k < nch - 1)
        def _():
            nxt = pl.ds((cbase + k + 1) * GPC, GPC)
            pltpu.async_copy(row_hbm.at[nxt, :], row_q, sem_i)
            pltpu.async_copy(col_hbm.at[nxt, :], col_q, sem_i)

        _drain(1, row_hbm.at[pl.ds(0, GPC), :].bitcast(jnp.float32), v0p,
               sem_g)
        _drain(1, row_hbm.at[pl.ds(0, GPC), :].bitcast(jnp.float32), v1p,
               sem_g)

        for j in range(GPC):
            pltpu.async_copy(v0p.at[j], acc0_sh.at[col_p.at[j]], sem_s,
                             add=True)
        for j in range(GPC):
            pltpu.async_copy(v1p.at[j], acc1_sh.at[col_p.at[j]], sem_s,
                             add=True)

    def chunk_body(k, carry):
        @pl.when(k % 2 == 0)
        def _():
            step(k, row_a, col_a, v0a, v1a, row_b, col_b, v0b, v1b)

        @pl.when(k % 2 == 1)
        def _():
            step(k, row_b, col_b, v0b, v1b, row_a, col_a, v0a, v1a)

        return carry

    lax.fori_loop(0, nch, chunk_body, 0)
    _drain(1, row_hbm.at[pl.ds(0, GPC), :].bitcast(jnp.float32), v0a, sem_s)
    _drain(1, row_hbm.at[pl.ds(0, GPC), :].bitcast(jnp.float32), v1a, sem_s)
    plsc.subcore_barrier()
    pltpu.sync_copy(acc0_sh.at[sl], acc0_hbm.at[c, sl])
    pltpu.sync_copy(acc1_sh.at[sl], acc1_hbm.at[c, sl])


def _dense_body(x_ref, w_ref, degp_ref, g0_ref, g1_ref, dinv_ref):
    deg = degp_ref[0] + degp_ref[1] + 1.0
    dinv = lax.rsqrt(deg)
    dinv_ref[...] = dinv
    x0, x1, x2 = x_ref[0], x_ref[1], x_ref[2]
    g0_ref[...] = (x0 * w_ref[0, 0] + x1 * w_ref[1, 0] + x2 * w_ref[2, 0]) * dinv
    g1_ref[...] = (x0 * w_ref[0, 1] + x1 * w_ref[1, 1] + x2 * w_ref[2, 1]) * dinv


_R = N_PAD // LANE
_dense = pl.pallas_call(
    _dense_body,
    out_shape=(jax.ShapeDtypeStruct((_R, LANE), jnp.float32),
               jax.ShapeDtypeStruct((_R, LANE), jnp.float32),
               jax.ShapeDtypeStruct((_R, LANE), jnp.float32)),
    in_specs=[pl.BlockSpec(memory_space=pltpu.VMEM),
              pl.BlockSpec(memory_space=pltpu.SMEM),
              pl.BlockSpec(memory_space=pltpu.VMEM)],
    out_specs=(pl.BlockSpec(memory_space=pltpu.VMEM),
               pl.BlockSpec(memory_space=pltpu.VMEM),
               pl.BlockSpec(memory_space=pltpu.VMEM)),
)


def _combine_body(acc0_ref, acc1_ref, g0_ref, g1_ref, dinv_ref, b_ref, o_ref):
    dinv = dinv_ref[...]
    o_ref[0] = dinv * (acc0_ref[0] + acc0_ref[1] + g0_ref[...]) + b_ref[0]
    o_ref[1] = dinv * (acc1_ref[0] + acc1_ref[1] + g1_ref[...]) + b_ref[1]


_combine = pl.pallas_call(
    _combine_body,
    out_shape=jax.ShapeDtypeStruct((2, _R, LANE), jnp.float32),
    in_specs=[pl.BlockSpec(memory_space=pltpu.VMEM),
              pl.BlockSpec(memory_space=pltpu.VMEM),
              pl.BlockSpec(memory_space=pltpu.VMEM),
              pl.BlockSpec(memory_space=pltpu.VMEM),
              pl.BlockSpec(memory_space=pltpu.VMEM),
              pl.BlockSpec(memory_space=pltpu.SMEM)],
    out_specs=pl.BlockSpec(memory_space=pltpu.VMEM),
)


def kernel(x, edge_index, W, b):
    ei = edge_index.astype(jnp.int32)
    row = ei[0].reshape(N_EDGES // GROUP, GROUP)
    col = ei[1].reshape(N_EDGES // GROUP, GROUP)
    ones_g = jnp.ones((GROUP,), jnp.float32)
    zeros_s = jnp.zeros((SUB,), jnp.float32)

    degp = _deg_kernel(col, ones_g, zeros_s)

    x_pad = jnp.pad(x.astype(jnp.float32).T, ((0, 0), (0, N_PAD - N_NODES)))
    x3 = x_pad.reshape(3, _R, LANE)
    g0, g1, dinv = _dense(x3, W.astype(jnp.float32),
                          degp.reshape(NC, _R, LANE))

    acc0, acc1 = _msg_kernel(row, col,
                             g0.reshape(N_PAD), g1.reshape(N_PAD), zeros_s)

    outT = _combine(acc0.reshape(NC, _R, LANE), acc1.reshape(NC, _R, LANE),
                    g0, g1, dinv, b.astype(jnp.float32))
    return outT.reshape(2, N_PAD)[:, :N_NODES].T
